# baseline (speedup 1.0000x reference)
import jax
import jax.numpy as jnp
from jax import lax
from jax.experimental import pallas as pl
from jax.experimental.pallas import tpu as pltpu
from jax.experimental.pallas import tpu_sc as plsc

NC, NS, LANES = 2, 16, 16
NW = NC * NS
NB = 4096
NT = 50
D = 64
CH = 128
NCH = NT
R = 5
MAP_AHEAD = 3


def _body(text_hbm, tw_hbm, fw_hbm, tmap_hbm, fmap_hbm, out_hbm,
          tok_v, tidx_v, fidx_v, fw_v, rows_a, tr_b,
          sem_row, sem_out, sem_map):
    wid = lax.axis_index("s") * NC + lax.axis_index("c")

    pltpu.sync_copy(fw_hbm, fw_v)
    pltpu.sync_copy(text_hbm.at[:, pl.ds(wid * CH, CH)], tok_v)

    def map_copies(c, s):
        return (
            pltpu.make_async_copy(tmap_hbm.at[tok_v.at[c]], tidx_v.at[c],
                                  sem_map.at[s]),
            pltpu.make_async_copy(fmap_hbm.at[tok_v.at[c]], fidx_v.at[c],
                                  sem_map.at[s]),
        )

    def row_copies(c, s):
        return (
            pltpu.make_async_copy(tw_hbm.at[tidx_v.at[c]], rows_a[s],
                                  sem_row.at[s]),
        )

    def out_copy(c, s):
        return pltpu.make_async_copy(
            tr_b[s], out_hbm.at[c, :, wid], sem_out.at[s])

    def issue(copies):
        for cp in copies:
            cp.start()

    def drain(copies):
        for cp in copies:
            cp.wait()

    def add_frozen(c, s):
        @pl.loop(0, CH // LANES)
        def _(g):
            fvec = fidx_v[c, pl.ds(g * LANES, LANES)]
            nfrozen = plsc.all_reduce_population_count(fvec != 0)

            @pl.when(nfrozen[0] > 0)
            def _():
                base = g * LANES
                for k in range(LANES):
                    f = fvec[k]

                    @pl.when(f != 0)
                    def _():
                        for cc in range(D // LANES):
                            sl = pl.ds(cc * LANES, LANES)
                            rows_a[s][base + k, sl] = (
                                rows_a[s][base + k, sl] + fw_v[f, sl])

    def transpose(s):
        iota = lax.iota(jnp.int32, LANES)
        dts = [(iota + c * LANES) // 8 for c in range(D // LANES)]
        dss = [(iota + c * LANES) % 8 for c in range(D // LANES)]
        zeros = jnp.zeros((LANES,), jnp.int32)
        GRP = 4

        @plsc.parallel_loop(0, CH // GRP, unroll=2)
        def _(g):
            vs = []
            for k in range(GRP):
                tok = g * GRP + k
                for c in range(D // LANES):
                    vs.append(rows_a[s][tok, pl.ds(c * LANES, LANES)])
            for k in range(GRP):
                tok = g * GRP + k
                blv = zeros + tok
                for c in range(D // LANES):
                    plsc.store_scatter(tr_b[s], [dts[c], dss[c], blv],
                                       vs[k * (D // LANES) + c])

    for c in range(MAP_AHEAD):
        issue(map_copies(c, c % R))
    drain(map_copies(0, 0))
    issue(row_copies(0, 0))

    @pl.loop(0, NCH, step=R)
    def _(i0):
        for b in range(R):
            i = i0 + b
            drain(row_copies(i, b))
            add_frozen(i, b)

            @pl.when(i >= R)
            def _():
                out_copy(i - R, b).wait()

            transpose(b)
            out_copy(i, b).start()

            j = i + 1
            bj = (b + 1) % R

            @pl.when(j < NCH)
            def _():
                drain(map_copies(j, bj))
                issue(row_copies(j, bj))

            m = i + MAP_AHEAD
            bm = (b + MAP_AHEAD) % R

            @pl.when(m < NCH)
            def _():
                issue(map_copies(m, bm))

    for b in range(R):
        out_copy(NCH - R + b, b).wait()


_run = pl.kernel(
    _body,
    out_type=jax.ShapeDtypeStruct((NT, 8, NB // CH, 8, CH), jnp.float32),
    mesh=plsc.VectorSubcoreMesh(core_axis_name="c", subcore_axis_name="s"),
    compiler_params=pltpu.CompilerParams(use_tc_tiling_on_sc=False,
                                         needs_layout_passes=False),
    scratch_types=[
        pltpu.VMEM((NCH, CH), jnp.int32),
        pltpu.VMEM((NCH, CH), jnp.int32),
        pltpu.VMEM((NCH, CH), jnp.int32),
        pltpu.VMEM((65, D), jnp.float32),
        [pltpu.VMEM((CH, D), jnp.float32) for _ in range(R)],
        [pltpu.VMEM((8, 8, CH), jnp.float32) for _ in range(R)],
        pltpu.SemaphoreType.DMA((R,)),
        pltpu.SemaphoreType.DMA((R,)),
        pltpu.SemaphoreType.DMA((R,)),
    ],
)


@jax.jit
def kernel(text_input, trainable_weight, frozen_weight, trainable_map,
           frozen_map):
    out5d = _run(text_input.T, trainable_weight, frozen_weight,
                 trainable_map, frozen_map)
    return out5d.transpose(2, 4, 0, 1, 3).reshape(NB, NT, D)


# speedup vs baseline: 1.3115x; 1.3115x over previous
import jax
import jax.numpy as jnp
from jax import lax
from jax.experimental import pallas as pl
from jax.experimental.pallas import tpu as pltpu
from jax.experimental.pallas import tpu_sc as plsc

NC, NS, LANES = 2, 16, 16
NW = NC * NS
B = 4096 * 50
D = 64
PER_W = B // NW
CH = 128
NCH = PER_W // CH
R = 5
MAP_AHEAD = 3


def _body(text_hbm, tw_hbm, fw_hbm, tmap_hbm, fmap_hbm, out_hbm,
          tok_v, tidx_v, fidx_v, fw_v, rows_a,
          sem_row, sem_out, sem_map):
    wid = lax.axis_index("s") * NC + lax.axis_index("c")

    pltpu.sync_copy(fw_hbm, fw_v)
    pltpu.sync_copy(text_hbm.at[wid], tok_v)

    def map_copies(c, s):
        return (
            pltpu.make_async_copy(tmap_hbm.at[tok_v.at[c]], tidx_v.at[c],
                                  sem_map.at[s]),
            pltpu.make_async_copy(fmap_hbm.at[tok_v.at[c]], fidx_v.at[c],
                                  sem_map.at[s]),
        )

    def row_copies(c, s):
        return (
            pltpu.make_async_copy(tw_hbm.at[tidx_v.at[c]], rows_a[s],
                                  sem_row.at[s]),
        )

    def out_copy(c, s):
        return pltpu.make_async_copy(
            rows_a[s], out_hbm.at[pl.ds(wid * PER_W + c * CH, CH)],
            sem_out.at[s])

    def issue(copies):
        for cp in copies:
            cp.start()

    def drain(copies):
        for cp in copies:
            cp.wait()

    def compute(c, s):
        @pl.loop(0, CH // LANES)
        def _(g):
            fvec = fidx_v[c, pl.ds(g * LANES, LANES)]
            nfrozen = plsc.all_reduce_population_count(fvec != 0)

            @pl.when(nfrozen[0] > 0)
            def _():
                base = g * LANES
                for k in range(LANES):
                    f = fvec[k]

                    @pl.when(f != 0)
                    def _():
                        for cc in range(D // LANES):
                            sl = pl.ds(cc * LANES, LANES)
                            rows_a[s][base + k, sl] = (
                                rows_a[s][base + k, sl] + fw_v[f, sl])

    for c in range(MAP_AHEAD):
        issue(map_copies(c, c % R))
    drain(map_copies(0, 0))
    issue(row_copies(0, 0))

    @pl.loop(0, NCH, step=R)
    def _(i0):
        for b in range(R):
            i = i0 + b
            drain(row_copies(i, b))
            compute(i, b)
            out_copy(i, b).start()

            j = i + 1
            bj = (b + 1) % R

            @pl.when(j < NCH)
            def _():
                drain(map_copies(j, bj))

                @pl.when(j >= R)
                def _():
                    out_copy(j - R, bj).wait()

                issue(row_copies(j, bj))

            m = i + MAP_AHEAD
            bm = (b + MAP_AHEAD) % R

            @pl.when(m < NCH)
            def _():
                issue(map_copies(m, bm))

    for b in range(R):
        out_copy(NCH - R + b, b).wait()


_run = pl.kernel(
    _body,
    out_type=jax.ShapeDtypeStruct((B, D), jnp.float32),
    mesh=plsc.VectorSubcoreMesh(core_axis_name="c", subcore_axis_name="s"),
    compiler_params=pltpu.CompilerParams(use_tc_tiling_on_sc=False,
                                         needs_layout_passes=False),
    scratch_types=[
        pltpu.VMEM((NCH, CH), jnp.int32),
        pltpu.VMEM((NCH, CH), jnp.int32),
        pltpu.VMEM((NCH, CH), jnp.int32),
        pltpu.VMEM((65, D), jnp.float32),
        [pltpu.VMEM((CH, D), jnp.float32) for _ in range(R)],
        pltpu.SemaphoreType.DMA((R,)),
        pltpu.SemaphoreType.DMA((R,)),
        pltpu.SemaphoreType.DMA((R,)),
    ],
)


@jax.jit
def kernel(text_input, trainable_weight, frozen_weight, trainable_map,
           frozen_map):
    text3d = text_input.reshape(NW, NCH, CH)
    out = _run(text3d, trainable_weight, frozen_weight, trainable_map,
               frozen_map)
    return out.reshape(text_input.shape[0], text_input.shape[1], D)


# speedup vs baseline: 1.3204x; 1.0068x over previous
import jax
import jax.numpy as jnp
from jax import lax
from jax.experimental import pallas as pl
from jax.experimental.pallas import tpu as pltpu
from jax.experimental.pallas import tpu_sc as plsc

NC, NS, LANES = 2, 16, 16
NW = NC * NS
B = 4096 * 50
D = 64
PER_W = B // NW
CH = 128
NCH = PER_W // CH
R = 5
MAP_AHEAD = 3


def _body(text_hbm, tw_hbm, fw_hbm, tmap_hbm, fmap_hbm, out_hbm,
          tok_v, tidx_v, fidx_v, fw_v, rows_a, tmap_sh, fmap_sh,
          sem_row, sem_out, sem_map):
    sid = lax.axis_index("s")
    wid = sid * NC + lax.axis_index("c")

    @pl.when(sid == 0)
    def _():
        pltpu.sync_copy(tmap_hbm, tmap_sh)
        pltpu.sync_copy(fmap_hbm, fmap_sh)

    pltpu.sync_copy(fw_hbm, fw_v)
    pltpu.sync_copy(text_hbm.at[wid], tok_v)
    plsc.subcore_barrier()

    def map_copies(c, s):
        return (
            pltpu.make_async_copy(tmap_sh.at[tok_v.at[c]], tidx_v.at[c],
                                  sem_map.at[s]),
            pltpu.make_async_copy(fmap_sh.at[tok_v.at[c]], fidx_v.at[c],
                                  sem_map.at[s]),
        )

    def row_copies(c, s):
        return (
            pltpu.make_async_copy(tw_hbm.at[tidx_v.at[c]], rows_a[s],
                                  sem_row.at[s]),
        )

    def out_copy(c, s):
        return pltpu.make_async_copy(
            rows_a[s], out_hbm.at[pl.ds(wid * PER_W + c * CH, CH)],
            sem_out.at[s])

    def issue(copies):
        for cp in copies:
            cp.start()

    def drain(copies):
        for cp in copies:
            cp.wait()

    def compute(c, s):
        @pl.loop(0, CH // LANES)
        def _(g):
            fvec = fidx_v[c, pl.ds(g * LANES, LANES)]
            nfrozen = plsc.all_reduce_population_count(fvec != 0)

            @pl.when(nfrozen[0] > 0)
            def _():
                base = g * LANES
                for k in range(LANES):
                    f = fvec[k]

                    @pl.when(f != 0)
                    def _():
                        for cc in range(D // LANES):
                            sl = pl.ds(cc * LANES, LANES)
                            rows_a[s][base + k, sl] = (
                                rows_a[s][base + k, sl] + fw_v[f, sl])

    for c in range(MAP_AHEAD):
        issue(map_copies(c, c % R))
    drain(map_copies(0, 0))
    issue(row_copies(0, 0))

    @pl.loop(0, NCH, step=R)
    def _(i0):
        for b in range(R):
            i = i0 + b
            drain(row_copies(i, b))
            compute(i, b)
            out_copy(i, b).start()

            j = i + 1
            bj = (b + 1) % R

            @pl.when(j < NCH)
            def _():
                drain(map_copies(j, bj))

                @pl.when(j >= R)
                def _():
                    out_copy(j - R, bj).wait()

                issue(row_copies(j, bj))

            m = i + MAP_AHEAD
            bm = (b + MAP_AHEAD) % R

            @pl.when(m < NCH)
            def _():
                issue(map_copies(m, bm))

    for b in range(R):
        out_copy(NCH - R + b, b).wait()


_run = pl.kernel(
    _body,
    out_type=jax.ShapeDtypeStruct((B, D), jnp.float32),
    mesh=plsc.VectorSubcoreMesh(core_axis_name="c", subcore_axis_name="s"),
    compiler_params=pltpu.CompilerParams(use_tc_tiling_on_sc=False,
                                         needs_layout_passes=False),
    scratch_types=[
        pltpu.VMEM((NCH, CH), jnp.int32),
        pltpu.VMEM((NCH, CH), jnp.int32),
        pltpu.VMEM((NCH, CH), jnp.int32),
        pltpu.VMEM((65, D), jnp.float32),
        [pltpu.VMEM((CH, D), jnp.float32) for _ in range(R)],
        pltpu.VMEM_SHARED((100000,), jnp.int32),
        pltpu.VMEM_SHARED((100000,), jnp.int32),
        pltpu.SemaphoreType.DMA((R,)),
        pltpu.SemaphoreType.DMA((R,)),
        pltpu.SemaphoreType.DMA((R,)),
    ],
)


@jax.jit
def kernel(text_input, trainable_weight, frozen_weight, trainable_map,
           frozen_map):
    text3d = text_input.reshape(NW, NCH, CH)
    out = _run(text3d, trainable_weight, frozen_weight, trainable_map,
               frozen_map)
    return out.reshape(text_input.shape[0], text_input.shape[1], D)


# speedup vs baseline: 1.4611x; 1.1065x over previous
import jax
import jax.numpy as jnp
from jax import lax
from jax.experimental import pallas as pl
from jax.experimental.pallas import tpu as pltpu
from jax.experimental.pallas import tpu_sc as plsc

NC, NS, LANES = 2, 16, 16
NW = NC * NS
B = 4096 * 50
D = 64
PER_W = B // NW
CH = 128
NCH = PER_W // CH
R = 5
ROW_AHEAD = 2
MAP_AHEAD = 4


def _body(text_hbm, tw_hbm, fw_hbm, tmap_hbm, fmap_hbm, out_hbm,
          tok_v, tidx_v, fidx_v, fw_v, rows_a, tmap_sh, fmap_sh,
          sem_row, sem_out, sem_map):
    sid = lax.axis_index("s")
    wid = sid * NC + lax.axis_index("c")

    @pl.when(sid == 0)
    def _():
        pltpu.sync_copy(tmap_hbm, tmap_sh)
        pltpu.sync_copy(fmap_hbm, fmap_sh)

    pltpu.sync_copy(fw_hbm, fw_v)
    pltpu.sync_copy(text_hbm.at[wid], tok_v)
    plsc.subcore_barrier()

    def map_copies(c, s):
        return (
            pltpu.make_async_copy(tmap_sh.at[tok_v.at[c]], tidx_v.at[c],
                                  sem_map.at[s]),
            pltpu.make_async_copy(fmap_sh.at[tok_v.at[c]], fidx_v.at[c],
                                  sem_map.at[s]),
        )

    def row_copies(c, s):
        return (
            pltpu.make_async_copy(tw_hbm.at[tidx_v.at[c]], rows_a[s],
                                  sem_row.at[s]),
        )

    def out_copy(c, s):
        return pltpu.make_async_copy(
            rows_a[s], out_hbm.at[pl.ds(wid * PER_W + c * CH, CH)],
            sem_out.at[s])

    def issue(copies):
        for cp in copies:
            cp.start()

    def drain(copies):
        for cp in copies:
            cp.wait()

    def compute(c, s):
        @pl.loop(0, CH // LANES)
        def _(g):
            fvec = fidx_v[c, pl.ds(g * LANES, LANES)]
            nfrozen = plsc.all_reduce_population_count(fvec != 0)

            @pl.when(nfrozen[0] > 0)
            def _():
                base = g * LANES
                for k in range(LANES):
                    f = fvec[k]

                    @pl.when(f != 0)
                    def _():
                        for cc in range(D // LANES):
                            sl = pl.ds(cc * LANES, LANES)
                            rows_a[s][base + k, sl] = (
                                rows_a[s][base + k, sl] + fw_v[f, sl])

    for c in range(MAP_AHEAD):
        issue(map_copies(c, c % R))
    for c in range(ROW_AHEAD):
        drain(map_copies(c, c % R))
        issue(row_copies(c, c % R))

    @pl.loop(0, NCH, step=R)
    def _(i0):
        for b in range(R):
            i = i0 + b
            drain(row_copies(i, b))
            compute(i, b)
            out_copy(i, b).start()

            j = i + ROW_AHEAD
            bj = (b + ROW_AHEAD) % R

            @pl.when(j < NCH)
            def _():
                drain(map_copies(j, bj))

                @pl.when(j >= R)
                def _():
                    out_copy(j - R, bj).wait()

                issue(row_copies(j, bj))

            m = i + MAP_AHEAD
            bm = (b + MAP_AHEAD) % R

            @pl.when(m < NCH)
            def _():
                issue(map_copies(m, bm))

    for b in range(R):
        out_copy(NCH - R + b, b).wait()


_run = pl.kernel(
    _body,
    out_type=jax.ShapeDtypeStruct((B, D), jnp.float32),
    mesh=plsc.VectorSubcoreMesh(core_axis_name="c", subcore_axis_name="s"),
    compiler_params=pltpu.CompilerParams(use_tc_tiling_on_sc=False,
                                         needs_layout_passes=False),
    scratch_types=[
        pltpu.VMEM((NCH, CH), jnp.int32),
        pltpu.VMEM((NCH, CH), jnp.int32),
        pltpu.VMEM((NCH, CH), jnp.int32),
        pltpu.VMEM((65, D), jnp.float32),
        [pltpu.VMEM((CH, D), jnp.float32) for _ in range(R)],
        pltpu.VMEM_SHARED((100000,), jnp.int32),
        pltpu.VMEM_SHARED((100000,), jnp.int32),
        pltpu.SemaphoreType.DMA((R,)),
        pltpu.SemaphoreType.DMA((R,)),
        pltpu.SemaphoreType.DMA((R,)),
    ],
)


@jax.jit
def kernel(text_input, trainable_weight, frozen_weight, trainable_map,
           frozen_map):
    text3d = text_input.reshape(NW, NCH, CH)
    out = _run(text3d, trainable_weight, frozen_weight, trainable_map,
               frozen_map)
    return out.reshape(text_input.shape[0], text_input.shape[1], D)


# speedup vs baseline: 1.4881x; 1.0185x over previous
import jax
import jax.numpy as jnp
from jax import lax
from jax.experimental import pallas as pl
from jax.experimental.pallas import tpu as pltpu
from jax.experimental.pallas import tpu_sc as plsc

NC, NS, LANES = 2, 16, 16
NW = NC * NS
B = 4096 * 50
D = 64
PER_W = B // NW
CH = 128
NCH = PER_W // CH
R = 5
ROW_AHEAD = 3
MAP_AHEAD = 5


def _body(text_hbm, tw_hbm, fw_hbm, tmap_hbm, fmap_hbm, out_hbm,
          tok_v, tidx_v, fidx_v, fw_v, rows_a, tmap_sh, fmap_sh,
          sem_row, sem_out, sem_map):
    sid = lax.axis_index("s")
    wid = sid * NC + lax.axis_index("c")

    @pl.when(sid == 0)
    def _():
        pltpu.sync_copy(tmap_hbm, tmap_sh)
        pltpu.sync_copy(fmap_hbm, fmap_sh)

    pltpu.sync_copy(fw_hbm, fw_v)
    pltpu.sync_copy(text_hbm.at[wid], tok_v)
    plsc.subcore_barrier()

    def map_copies(c, s):
        return (
            pltpu.make_async_copy(tmap_sh.at[tok_v.at[c]], tidx_v.at[c],
                                  sem_map.at[s]),
            pltpu.make_async_copy(fmap_sh.at[tok_v.at[c]], fidx_v.at[c],
                                  sem_map.at[s]),
        )

    def row_copies(c, s):
        return (
            pltpu.make_async_copy(tw_hbm.at[tidx_v.at[c]], rows_a[s],
                                  sem_row.at[s]),
        )

    def out_copy(c, s):
        return pltpu.make_async_copy(
            rows_a[s], out_hbm.at[pl.ds(wid * PER_W + c * CH, CH)],
            sem_out.at[s])

    def issue(copies):
        for cp in copies:
            cp.start()

    def drain(copies):
        for cp in copies:
            cp.wait()

    def compute(c, s):
        @pl.loop(0, CH // LANES)
        def _(g):
            fvec = fidx_v[c, pl.ds(g * LANES, LANES)]
            nfrozen = plsc.all_reduce_population_count(fvec != 0)

            @pl.when(nfrozen[0] > 0)
            def _():
                base = g * LANES
                for k in range(LANES):
                    f = fvec[k]

                    @pl.when(f != 0)
                    def _():
                        for cc in range(D // LANES):
                            sl = pl.ds(cc * LANES, LANES)
                            rows_a[s][base + k, sl] = (
                                rows_a[s][base + k, sl] + fw_v[f, sl])

    for c in range(MAP_AHEAD):
        issue(map_copies(c, c % R))
    for c in range(ROW_AHEAD):
        drain(map_copies(c, c % R))
        issue(row_copies(c, c % R))

    @pl.loop(0, NCH, step=R)
    def _(i0):
        for b in range(R):
            i = i0 + b
            drain(row_copies(i, b))
            compute(i, b)
            out_copy(i, b).start()

            j = i + ROW_AHEAD
            bj = (b + ROW_AHEAD) % R

            @pl.when(j < NCH)
            def _():
                drain(map_copies(j, bj))

                @pl.when(j >= R)
                def _():
                    out_copy(j - R, bj).wait()

                issue(row_copies(j, bj))

            m = i + MAP_AHEAD
            bm = (b + MAP_AHEAD) % R

            @pl.when(m < NCH)
            def _():
                issue(map_copies(m, bm))

    for b in range(R):
        out_copy(NCH - R + b, b).wait()


_run = pl.kernel(
    _body,
    out_type=jax.ShapeDtypeStruct((B, D), jnp.float32),
    mesh=plsc.VectorSubcoreMesh(core_axis_name="c", subcore_axis_name="s"),
    compiler_params=pltpu.CompilerParams(use_tc_tiling_on_sc=False,
                                         needs_layout_passes=False),
    scratch_types=[
        pltpu.VMEM((NCH, CH), jnp.int32),
        pltpu.VMEM((NCH, CH), jnp.int32),
        pltpu.VMEM((NCH, CH), jnp.int32),
        pltpu.VMEM((65, D), jnp.float32),
        [pltpu.VMEM((CH, D), jnp.float32) for _ in range(R)],
        pltpu.VMEM_SHARED((100000,), jnp.int32),
        pltpu.VMEM_SHARED((100000,), jnp.int32),
        pltpu.SemaphoreType.DMA((R,)),
        pltpu.SemaphoreType.DMA((R,)),
        pltpu.SemaphoreType.DMA((R,)),
    ],
)


@jax.jit
def kernel(text_input, trainable_weight, frozen_weight, trainable_map,
           frozen_map):
    text3d = text_input.reshape(NW, NCH, CH)
    out = _run(text3d, trainable_weight, frozen_weight, trainable_map,
               frozen_map)
    return out.reshape(text_input.shape[0], text_input.shape[1], D)
